# baseline (speedup 1.0000x reference)
import functools

import jax
import jax.numpy as jnp
from jax import lax
from jax.experimental import pallas as pl
from jax.experimental.pallas import tpu as pltpu
from jax.experimental.pallas import tpu_sc as plsc

N_NODES = 10000
E = 320000
E2 = 2 * E
D = 128

NC = 2
NS = 16
NW = NC * NS
CB = 128


def _pad_to(n, m):
    return ((n + m - 1) // m) * m


STRIPS = 2
P_EDGE = _pad_to(E2, STRIPS * NW * CB * 2)
P_CLF = _pad_to(E, STRIPS * NW * CB * 2)
S_EDGE = P_EDGE // STRIPS
S_CLF = P_CLF // STRIPS


def _dot(a, b):
    return jax.lax.dot_general(
        a.astype(jnp.bfloat16), b.astype(jnp.bfloat16),
        dimension_numbers=(((1,), (0,)), ((), ())),
        preferred_element_type=jnp.float32)


def _rowsum(t, o):
    th = t.astype(jnp.bfloat16)
    tl = (t - th.astype(jnp.float32)).astype(jnp.bfloat16)
    d = functools.partial(
        jax.lax.dot_general,
        dimension_numbers=(((1,), (0,)), ((), ())),
        preferred_element_type=jnp.float32)
    return d(th, o) + d(tl, o)


def _ln_relu(t, g, bt, o):
    mu = _rowsum(t, o)
    q = _rowsum(t * t, o)
    var = q - mu * mu
    rstd = lax.rsqrt(var + 1e-5)
    return jnp.maximum((t - mu) * rstd * g + bt, 0.0)


def _ln(t, g, bt):
    mu = jnp.mean(t, axis=-1, keepdims=True)
    var = jnp.mean((t - mu) ** 2, axis=-1, keepdims=True)
    return (t - mu) * lax.rsqrt(var + 1e-5) * g + bt


def _enc_body(x_ref, w1, b1, w2, b2, wa, wb, h_ref, a_ref, b_ref):
    h = jnp.maximum(_dot(x_ref[...], w1[...]) + b1[...], 0.0)
    h = jnp.maximum(_dot(h, w2[...]) + b2[...], 0.0)
    h_ref[...] = h
    a_ref[...] = _dot(h, wa[...])
    b_ref[...] = _dot(h, wb[...])


def _enc_call(x, W1, b1, W2, b2, Wa, Wb):
    RB = 1000
    wspec = pl.BlockSpec((D, D), lambda i: (0, 0))
    bspec = pl.BlockSpec((1, D), lambda i: (0, 0))
    rspec = pl.BlockSpec((RB, D), lambda i: (i, 0))
    return pl.pallas_call(
        _enc_body,
        grid=(N_NODES // RB,),
        in_specs=[rspec, wspec, bspec, wspec, bspec, wspec, wspec],
        out_specs=[rspec, rspec, rspec],
        out_shape=[jax.ShapeDtypeStruct((N_NODES, D), jnp.float32)] * 3,
    )(x, W1, b1, W2, b2, Wa, Wb)


def _make_edge_body(limit):
    def _edge_body(ag_ref, bg_ref, b1, g1, t1, w2, b2, g2, t2,
                   w3, b3, g3, t3, w4, b4, g4, t4, e_ref):
        RB = ag_ref.shape[0]
        t = ag_ref[...] + bg_ref[...] + b1[...]
        t = jnp.maximum(_ln(t, g1[...], t1[...]), 0.0)
        t = jnp.maximum(_ln(_dot(t, w2[...]) + b2[...], g2[...], t2[...]), 0.0)
        t = jnp.maximum(_ln(_dot(t, w3[...]) + b3[...], g3[...], t3[...]), 0.0)
        t = jnp.maximum(_ln(_dot(t, w4[...]) + b4[...], g4[...], t4[...]), 0.0)
        row = (pl.program_id(0) * RB
               + lax.broadcasted_iota(jnp.int32, (RB, 1), 0))
        e_ref[...] = jnp.where(row < limit, t, 0.0)
    return _edge_body


def _edge_call(ag, bg, ep, base):
    RB = 2048
    P = ag.shape[0]
    wspec = pl.BlockSpec((D, D), lambda i: (0, 0))
    bspec = pl.BlockSpec((1, D), lambda i: (0, 0))
    rspec = pl.BlockSpec((RB, D), lambda i: (i, 0))
    (w1_, b1, g1, t1), (w2, b2, g2, t2), (w3, b3, g3, t3), (w4, b4, g4, t4) = ep
    args = [ag, bg, b1, g1, t1, w2, b2, g2, t2,
            w3, b3, g3, t3, w4, b4, g4, t4]
    specs = [rspec, rspec, bspec, bspec, bspec, wspec, bspec, bspec,
             bspec, wspec, bspec, bspec, bspec, wspec, bspec, bspec, bspec]
    return pl.pallas_call(
        _make_edge_body(E2 - base),
        grid=(P // RB,),
        in_specs=specs,
        out_specs=rspec,
        out_shape=jax.ShapeDtypeStruct((P, D), jnp.float32),
    )(*args)


def _node_body(h_ref, p0_ref, p1_ref, p2_ref, p3_ref, wt, wb, b1, g1, t1,
               w2, b2, g2, t2, w3, b3, g3, t3, w4, b4, g4, t4, wct, wcb,
               a2_ref, b2_ref):
    h = h_ref[...]
    aggr = (p0_ref[...] + p1_ref[...]) + (p2_ref[...] + p3_ref[...])
    t = _dot(h, wt[...]) + _dot(aggr, wb[...]) + b1[...]
    t = jnp.maximum(_ln(t, g1[...], t1[...]), 0.0)
    t = jnp.maximum(_ln(_dot(t, w2[...]) + b2[...], g2[...], t2[...]), 0.0)
    t = jnp.maximum(_ln(_dot(t, w3[...]) + b3[...], g3[...], t3[...]), 0.0)
    t = jnp.maximum(_ln(_dot(t, w4[...]) + b4[...], g4[...], t4[...]), 0.0)
    h2 = t + h
    a2_ref[...] = _dot(h2, wct[...])
    b2_ref[...] = _dot(h2, wcb[...])


def _node_call(h, parts, np_, Wct, Wcb):
    RB = 1000
    wspec = pl.BlockSpec((D, D), lambda i: (0, 0))
    bspec = pl.BlockSpec((1, D), lambda i: (0, 0))
    rspec = pl.BlockSpec((RB, D), lambda i: (i, 0))
    (w1, b1, g1, t1), (w2, b2, g2, t2), (w3, b3, g3, t3), (w4, b4, g4, t4) = np_
    args = [h, *parts, w1[:D], w1[D:], b1, g1, t1, w2, b2, g2, t2,
            w3, b3, g3, t3, w4, b4, g4, t4, Wct, Wcb]
    specs = [rspec, rspec, rspec, rspec, rspec, wspec, wspec,
             bspec, bspec, bspec,
             wspec, bspec, bspec, bspec, wspec, bspec, bspec, bspec,
             wspec, bspec, bspec, bspec, wspec, wspec]
    return pl.pallas_call(
        _node_body,
        grid=(N_NODES // RB,),
        in_specs=specs,
        out_specs=[rspec, rspec],
        out_shape=[jax.ShapeDtypeStruct((N_NODES, D), jnp.float32)] * 2,
    )(*args)


def _clf_body(ag_ref, bg_ref, b1, w2r, b2, o_ref):
    t = jnp.maximum(ag_ref[...] + bg_ref[...] + b1[...], 0.0)
    tb = t.astype(jnp.bfloat16).astype(jnp.float32)
    wb = w2r[...].astype(jnp.bfloat16).astype(jnp.float32)
    y = jnp.sum(tb * wb, axis=1) + b2[0, 0]
    o_ref[...] = y[None, None, :]


def _clf_call(ag, bg, b1, W2, b2):
    RB = 2048
    P = ag.shape[0]
    rspec = pl.BlockSpec((RB, D), lambda i: (i, 0))
    return pl.pallas_call(
        _clf_body,
        grid=(P // RB,),
        in_specs=[rspec, rspec, pl.BlockSpec((1, D), lambda i: (0, 0)),
                  pl.BlockSpec((1, D), lambda i: (0, 0)),
                  pl.BlockSpec((1, 1), lambda i: (0, 0))],
        out_specs=pl.BlockSpec((1, 1, RB), lambda i: (i, 0, 0)),
        out_shape=jax.ShapeDtypeStruct((P // RB, 1, RB), jnp.float32),
    )(ag, bg, b1, W2.reshape(1, -1), b2)


def _gather_pair_call(P, a, b, ia, ib):
    per_w = P // NW
    n_chunks = per_w // CB
    dt = a.dtype
    W = a.shape[1]
    mesh = plsc.VectorSubcoreMesh(core_axis_name="c", subcore_axis_name="s")

    @functools.partial(
        pl.kernel,
        out_type=(jax.ShapeDtypeStruct((P, W), dt),
                  jax.ShapeDtypeStruct((P, W), dt)),
        mesh=mesh,
        scratch_types=[pltpu.VMEM((2, CB), jnp.int32),
                       pltpu.VMEM((2, CB), jnp.int32),
                       pltpu.VMEM((CB, W), dt),
                       pltpu.VMEM((CB, W), dt),
                       pltpu.VMEM((CB, W), dt),
                       pltpu.VMEM((CB, W), dt),
                       pltpu.SemaphoreType.DMA,
                       pltpu.SemaphoreType.DMA,
                       pltpu.SemaphoreType.DMA,
                       pltpu.SemaphoreType.DMA],
    )
    def gather2(a_hbm, b_hbm, ia_hbm, ib_hbm, oa_hbm, ob_hbm,
                ia_v, ib_v, ra0, rb0, ra1, rb1, sa0, sb0, sa1, sb1):
        wid = lax.axis_index("s") * NC + lax.axis_index("c")
        base0 = wid * per_w

        @pl.loop(0, n_chunks, step=2)
        def _(i):
            b0 = base0 + i * CB
            b1 = b0 + CB
            pltpu.sync_copy(ia_hbm.at[pl.ds(b0, CB)], ia_v.at[0])
            pltpu.sync_copy(ib_hbm.at[pl.ds(b0, CB)], ib_v.at[0])
            pltpu.sync_copy(ia_hbm.at[pl.ds(b1, CB)], ia_v.at[1])
            pltpu.sync_copy(ib_hbm.at[pl.ds(b1, CB)], ib_v.at[1])
            ca0 = pltpu.async_copy(a_hbm.at[ia_v.at[0]], ra0, sa0)
            cb0 = pltpu.async_copy(b_hbm.at[ib_v.at[0]], rb0, sb0)
            ca1 = pltpu.async_copy(a_hbm.at[ia_v.at[1]], ra1, sa1)
            cb1 = pltpu.async_copy(b_hbm.at[ib_v.at[1]], rb1, sb1)
            ca0.wait()
            cb0.wait()
            pltpu.sync_copy(ra0, oa_hbm.at[pl.ds(b0, CB)])
            pltpu.sync_copy(rb0, ob_hbm.at[pl.ds(b0, CB)])
            ca1.wait()
            cb1.wait()
            pltpu.sync_copy(ra1, oa_hbm.at[pl.ds(b1, CB)])
            pltpu.sync_copy(rb1, ob_hbm.at[pl.ds(b1, CB)])

    return gather2(a, b, ia, ib)


def _scatter_add_call(e, idx):
    per_core = e.shape[0] // NC
    per_sub = per_core // NS
    n_chunks = per_sub // CB
    RPS = 624
    LAST = N_NODES - (NS - 1) * RPS
    ZR = 16
    mesh = plsc.VectorSubcoreMesh(core_axis_name="c", subcore_axis_name="s")

    @functools.partial(
        pl.kernel,
        out_type=jax.ShapeDtypeStruct((NC, N_NODES, D), jnp.float32),
        mesh=mesh,
        scratch_types=[pltpu.VMEM((CB,), jnp.int32),
                       pltpu.VMEM((CB, D), jnp.float32),
                       pltpu.VMEM((ZR, D), jnp.float32),
                       pltpu.VMEM_SHARED((N_NODES, D), jnp.float32)],
    )
    def scatter_add(e_hbm, idx_hbm, out_hbm, idx_v, rows_v, z_v, acc_sh):
        cid = lax.axis_index("c")
        sid = lax.axis_index("s")

        @pl.loop(0, ZR)
        def _(r):
            @pl.loop(0, D // 16)
            def _(k):
                z_v[r, pl.ds(k * 16, 16)] = jnp.zeros((16,), jnp.float32)

        @pl.loop(0, LAST // ZR)
        def _(j):
            pltpu.sync_copy(z_v, acc_sh.at[pl.ds(sid * RPS + j * ZR, ZR)])

        plsc.subcore_barrier()
        base0 = cid * per_core + sid * per_sub

        @pl.loop(0, n_chunks)
        def _(i):
            base = base0 + i * CB
            pltpu.sync_copy(idx_hbm.at[pl.ds(base, CB)], idx_v)
            pltpu.sync_copy(e_hbm.at[pl.ds(base, CB)], rows_v)
            pltpu.sync_copy(rows_v, acc_sh.at[idx_v], add=True)

        plsc.subcore_barrier()

        @pl.when(sid != NS - 1)
        def _():
            pltpu.sync_copy(acc_sh.at[pl.ds(sid * RPS, RPS)],
                            out_hbm.at[cid].at[pl.ds(sid * RPS, RPS)])

        @pl.when(sid == NS - 1)
        def _():
            pltpu.sync_copy(acc_sh.at[pl.ds((NS - 1) * RPS, LAST)],
                            out_hbm.at[cid].at[pl.ds((NS - 1) * RPS, LAST)])

    return scatter_add(e, idx)


def kernel(x, edge_index, enc_params, edge_params, node_params, clf_params):
    ei0, ei1 = edge_index[0], edge_index[1]
    send = jnp.concatenate([ei0, ei1])
    recv = jnp.concatenate([ei1, ei0])
    send_p = jnp.pad(send, (0, P_EDGE - E2))
    recv_p = jnp.pad(recv, (0, P_EDGE - E2))
    ei0_p = jnp.pad(ei0, (0, P_CLF - E))
    ei1_p = jnp.pad(ei1, (0, P_CLF - E))

    (W1e, b1e, _, _), (W2e, b2e, _, _) = enc_params
    We1 = edge_params[0][0]
    row = lambda v: v.reshape(1, -1)

    ep = [(w, row(b), row(g), row(bt)) for (w, b, g, bt) in edge_params]
    npar = [(w, row(b), row(g), row(bt)) for (w, b, g, bt) in node_params]
    (Wc1, bc1, _, _), (Wc2, bc2, _, _) = clf_params

    h, A, B = _enc_call(x, W1e, row(b1e), W2e, row(b2e), We1[:D], We1[D:])

    parts = []
    for s in range(STRIPS):
        sl = slice(s * S_EDGE, (s + 1) * S_EDGE)
        Ag, Bg = _gather_pair_call(S_EDGE, A, B, send_p[sl], recv_p[sl])
        e = _edge_call(Ag, Bg, ep, s * S_EDGE)
        ps = _scatter_add_call(e, recv_p[sl])
        parts += [ps[0], ps[1]]

    A2, B2 = _node_call(h, parts, npar, Wc1[:D], Wc1[D:])

    outs = []
    for s in range(STRIPS):
        sl = slice(s * S_CLF, (s + 1) * S_CLF)
        A2g, B2g = _gather_pair_call(S_CLF, A2, B2, ei0_p[sl], ei1_p[sl])
        outs.append(_clf_call(A2g, B2g, row(bc1), Wc2, bc2.reshape(1, 1)))
    out = jnp.concatenate([o.reshape(-1) for o in outs])
    return out[:E]


# speedup vs baseline: 1.0997x; 1.0997x over previous
import functools

import jax
import jax.numpy as jnp
from jax import lax
from jax.experimental import pallas as pl
from jax.experimental.pallas import tpu as pltpu
from jax.experimental.pallas import tpu_sc as plsc

N_NODES = 10000
E = 320000
E2 = 2 * E
D = 128

NC = 2
NS = 16
NW = NC * NS
CB = 128


def _pad_to(n, m):
    return ((n + m - 1) // m) * m


STRIPS = 2
P_EDGE = _pad_to(E2, STRIPS * NW * CB)
P_CLF = _pad_to(E, STRIPS * NW * CB)
S_EDGE = P_EDGE // STRIPS
S_CLF = P_CLF // STRIPS


def _dot(a, b):
    return jax.lax.dot_general(
        a.astype(jnp.bfloat16), b.astype(jnp.bfloat16),
        dimension_numbers=(((1,), (0,)), ((), ())),
        preferred_element_type=jnp.float32)


def _rowsum(t, o):
    th = t.astype(jnp.bfloat16)
    tl = (t - th.astype(jnp.float32)).astype(jnp.bfloat16)
    d = functools.partial(
        jax.lax.dot_general,
        dimension_numbers=(((1,), (0,)), ((), ())),
        preferred_element_type=jnp.float32)
    return d(th, o) + d(tl, o)


def _ln_relu(t, g, bt, o):
    mu = _rowsum(t, o)
    q = _rowsum(t * t, o)
    var = q - mu * mu
    rstd = lax.rsqrt(var + 1e-5)
    return jnp.maximum((t - mu) * rstd * g + bt, 0.0)


def _ln(t, g, bt):
    mu = jnp.mean(t, axis=-1, keepdims=True)
    var = jnp.mean((t - mu) ** 2, axis=-1, keepdims=True)
    return (t - mu) * lax.rsqrt(var + 1e-5) * g + bt


def _enc_body(x_ref, w1, b1, w2, b2, wa, wb, h_ref, a_ref, b_ref):
    h = jnp.maximum(_dot(x_ref[...], w1[...]) + b1[...], 0.0)
    h = jnp.maximum(_dot(h, w2[...]) + b2[...], 0.0)
    h_ref[...] = h
    a_ref[...] = _dot(h, wa[...])
    b_ref[...] = _dot(h, wb[...])


def _enc_call(x, W1, b1, W2, b2, Wa, Wb):
    RB = 1000
    wspec = pl.BlockSpec((D, D), lambda i: (0, 0))
    bspec = pl.BlockSpec((1, D), lambda i: (0, 0))
    rspec = pl.BlockSpec((RB, D), lambda i: (i, 0))
    return pl.pallas_call(
        _enc_body,
        grid=(N_NODES // RB,),
        in_specs=[rspec, wspec, bspec, wspec, bspec, wspec, wspec],
        out_specs=[rspec, rspec, rspec],
        out_shape=[jax.ShapeDtypeStruct((N_NODES, D), jnp.float32)] * 3,
    )(x, W1, b1, W2, b2, Wa, Wb)


def _make_edge_body(limit):
    def _edge_body(ag_ref, bg_ref, b1, g1, t1, w2, b2, g2, t2,
                   w3, b3, g3, t3, w4, b4, g4, t4, e_ref):
        RB = ag_ref.shape[0]
        t = ag_ref[...] + bg_ref[...] + b1[...]
        t = jnp.maximum(_ln(t, g1[...], t1[...]), 0.0)
        t = jnp.maximum(_ln(_dot(t, w2[...]) + b2[...], g2[...], t2[...]), 0.0)
        t = jnp.maximum(_ln(_dot(t, w3[...]) + b3[...], g3[...], t3[...]), 0.0)
        t = jnp.maximum(_ln(_dot(t, w4[...]) + b4[...], g4[...], t4[...]), 0.0)
        row = (pl.program_id(0) * RB
               + lax.broadcasted_iota(jnp.int32, (RB, 1), 0))
        e_ref[...] = jnp.where(row < limit, t, 0.0)
    return _edge_body


def _edge_call(ag, bg, ep, base):
    RB = 2048
    P = ag.shape[0]
    wspec = pl.BlockSpec((D, D), lambda i: (0, 0))
    bspec = pl.BlockSpec((1, D), lambda i: (0, 0))
    rspec = pl.BlockSpec((RB, D), lambda i: (i, 0))
    (w1_, b1, g1, t1), (w2, b2, g2, t2), (w3, b3, g3, t3), (w4, b4, g4, t4) = ep
    args = [ag, bg, b1, g1, t1, w2, b2, g2, t2,
            w3, b3, g3, t3, w4, b4, g4, t4]
    specs = [rspec, rspec, bspec, bspec, bspec, wspec, bspec, bspec,
             bspec, wspec, bspec, bspec, bspec, wspec, bspec, bspec, bspec]
    return pl.pallas_call(
        _make_edge_body(E2 - base),
        grid=(P // RB,),
        in_specs=specs,
        out_specs=rspec,
        out_shape=jax.ShapeDtypeStruct((P, D), jnp.float32),
    )(*args)


def _node_body(h_ref, p0_ref, p1_ref, p2_ref, p3_ref, wt, wb, b1, g1, t1,
               w2, b2, g2, t2, w3, b3, g3, t3, w4, b4, g4, t4, wct, wcb,
               a2_ref, b2_ref):
    h = h_ref[...]
    aggr = (p0_ref[...] + p1_ref[...]) + (p2_ref[...] + p3_ref[...])
    t = _dot(h, wt[...]) + _dot(aggr, wb[...]) + b1[...]
    t = jnp.maximum(_ln(t, g1[...], t1[...]), 0.0)
    t = jnp.maximum(_ln(_dot(t, w2[...]) + b2[...], g2[...], t2[...]), 0.0)
    t = jnp.maximum(_ln(_dot(t, w3[...]) + b3[...], g3[...], t3[...]), 0.0)
    t = jnp.maximum(_ln(_dot(t, w4[...]) + b4[...], g4[...], t4[...]), 0.0)
    h2 = t + h
    a2_ref[...] = _dot(h2, wct[...])
    b2_ref[...] = _dot(h2, wcb[...])


def _node_call(h, parts, np_, Wct, Wcb):
    RB = 1000
    wspec = pl.BlockSpec((D, D), lambda i: (0, 0))
    bspec = pl.BlockSpec((1, D), lambda i: (0, 0))
    rspec = pl.BlockSpec((RB, D), lambda i: (i, 0))
    (w1, b1, g1, t1), (w2, b2, g2, t2), (w3, b3, g3, t3), (w4, b4, g4, t4) = np_
    args = [h, *parts, w1[:D], w1[D:], b1, g1, t1, w2, b2, g2, t2,
            w3, b3, g3, t3, w4, b4, g4, t4, Wct, Wcb]
    specs = [rspec, rspec, rspec, rspec, rspec, wspec, wspec,
             bspec, bspec, bspec,
             wspec, bspec, bspec, bspec, wspec, bspec, bspec, bspec,
             wspec, bspec, bspec, bspec, wspec, wspec]
    return pl.pallas_call(
        _node_body,
        grid=(N_NODES // RB,),
        in_specs=specs,
        out_specs=[rspec, rspec],
        out_shape=[jax.ShapeDtypeStruct((N_NODES, D), jnp.float32)] * 2,
    )(*args)


def _clf_body(ag_ref, bg_ref, b1, w2r, b2, o_ref):
    t = jnp.maximum(ag_ref[...] + bg_ref[...] + b1[...], 0.0)
    tb = t.astype(jnp.bfloat16).astype(jnp.float32)
    wb = w2r[...].astype(jnp.bfloat16).astype(jnp.float32)
    y = jnp.sum(tb * wb, axis=1) + b2[0, 0]
    o_ref[...] = y[None, None, :]


def _clf_call(ag, bg, b1, W2, b2):
    RB = 2048
    P = ag.shape[0]
    rspec = pl.BlockSpec((RB, D), lambda i: (i, 0))
    return pl.pallas_call(
        _clf_body,
        grid=(P // RB,),
        in_specs=[rspec, rspec, pl.BlockSpec((1, D), lambda i: (0, 0)),
                  pl.BlockSpec((1, D), lambda i: (0, 0)),
                  pl.BlockSpec((1, 1), lambda i: (0, 0))],
        out_specs=pl.BlockSpec((1, 1, RB), lambda i: (i, 0, 0)),
        out_shape=jax.ShapeDtypeStruct((P // RB, 1, RB), jnp.float32),
    )(ag, bg, b1, W2.reshape(1, -1), b2)


def _gather_pair_call(P, a, b, ia, ib):
    per_w = P // NW
    n_chunks = per_w // CB
    dt = a.dtype
    W = a.shape[1]
    mesh = plsc.VectorSubcoreMesh(core_axis_name="c", subcore_axis_name="s")

    @functools.partial(
        pl.kernel,
        out_type=(jax.ShapeDtypeStruct((P, W), dt),
                  jax.ShapeDtypeStruct((P, W), dt)),
        mesh=mesh,
        scratch_types=[pltpu.VMEM((CB,), jnp.int32),
                       pltpu.VMEM((CB,), jnp.int32),
                       pltpu.VMEM((CB, W), dt),
                       pltpu.VMEM((CB, W), dt),
                       pltpu.SemaphoreType.DMA,
                       pltpu.SemaphoreType.DMA],
    )
    def gather2(a_hbm, b_hbm, ia_hbm, ib_hbm, oa_hbm, ob_hbm,
                ia_v, ib_v, ra_v, rb_v, sa, sb):
        wid = lax.axis_index("s") * NC + lax.axis_index("c")
        base0 = wid * per_w

        @pl.loop(0, n_chunks)
        def _(i):
            base = base0 + i * CB
            pltpu.sync_copy(ia_hbm.at[pl.ds(base, CB)], ia_v)
            pltpu.sync_copy(ib_hbm.at[pl.ds(base, CB)], ib_v)
            ca = pltpu.async_copy(a_hbm.at[ia_v], ra_v, sa)
            cb = pltpu.async_copy(b_hbm.at[ib_v], rb_v, sb)
            ca.wait()
            cb.wait()
            pltpu.sync_copy(ra_v, oa_hbm.at[pl.ds(base, CB)])
            pltpu.sync_copy(rb_v, ob_hbm.at[pl.ds(base, CB)])

    return gather2(a, b, ia, ib)


def _scatter_add_call(e, idx):
    per_core = e.shape[0] // NC
    per_sub = per_core // NS
    n_chunks = per_sub // CB
    RPS = 624
    LAST = N_NODES - (NS - 1) * RPS
    ZR = 16
    mesh = plsc.VectorSubcoreMesh(core_axis_name="c", subcore_axis_name="s")

    @functools.partial(
        pl.kernel,
        out_type=jax.ShapeDtypeStruct((NC, N_NODES, D), jnp.float32),
        mesh=mesh,
        scratch_types=[pltpu.VMEM((CB,), jnp.int32),
                       pltpu.VMEM((CB, D), jnp.float32),
                       pltpu.VMEM((ZR, D), jnp.float32),
                       pltpu.VMEM_SHARED((N_NODES, D), jnp.float32)],
    )
    def scatter_add(e_hbm, idx_hbm, out_hbm, idx_v, rows_v, z_v, acc_sh):
        cid = lax.axis_index("c")
        sid = lax.axis_index("s")

        @pl.loop(0, ZR)
        def _(r):
            @pl.loop(0, D // 16)
            def _(k):
                z_v[r, pl.ds(k * 16, 16)] = jnp.zeros((16,), jnp.float32)

        @pl.loop(0, LAST // ZR)
        def _(j):
            pltpu.sync_copy(z_v, acc_sh.at[pl.ds(sid * RPS + j * ZR, ZR)])

        plsc.subcore_barrier()
        base0 = cid * per_core + sid * per_sub

        @pl.loop(0, n_chunks)
        def _(i):
            base = base0 + i * CB
            pltpu.sync_copy(idx_hbm.at[pl.ds(base, CB)], idx_v)
            pltpu.sync_copy(e_hbm.at[pl.ds(base, CB)], rows_v)
            pltpu.sync_copy(rows_v, acc_sh.at[idx_v], add=True)

        plsc.subcore_barrier()

        @pl.when(sid != NS - 1)
        def _():
            pltpu.sync_copy(acc_sh.at[pl.ds(sid * RPS, RPS)],
                            out_hbm.at[cid].at[pl.ds(sid * RPS, RPS)])

        @pl.when(sid == NS - 1)
        def _():
            pltpu.sync_copy(acc_sh.at[pl.ds((NS - 1) * RPS, LAST)],
                            out_hbm.at[cid].at[pl.ds((NS - 1) * RPS, LAST)])

    return scatter_add(e, idx)


def kernel(x, edge_index, enc_params, edge_params, node_params, clf_params):
    ei0, ei1 = edge_index[0], edge_index[1]
    send = jnp.concatenate([ei0, ei1])
    recv = jnp.concatenate([ei1, ei0])
    send_p = jnp.pad(send, (0, P_EDGE - E2))
    recv_p = jnp.pad(recv, (0, P_EDGE - E2))
    ei0_p = jnp.pad(ei0, (0, P_CLF - E))
    ei1_p = jnp.pad(ei1, (0, P_CLF - E))

    (W1e, b1e, _, _), (W2e, b2e, _, _) = enc_params
    We1 = edge_params[0][0]
    row = lambda v: v.reshape(1, -1)

    ep = [(w, row(b), row(g), row(bt)) for (w, b, g, bt) in edge_params]
    npar = [(w, row(b), row(g), row(bt)) for (w, b, g, bt) in node_params]
    (Wc1, bc1, _, _), (Wc2, bc2, _, _) = clf_params

    h, A, B = _enc_call(x, W1e, row(b1e), W2e, row(b2e), We1[:D], We1[D:])

    parts = []
    for s in range(STRIPS):
        sl = slice(s * S_EDGE, (s + 1) * S_EDGE)
        Ag, Bg = _gather_pair_call(S_EDGE, A, B, send_p[sl], recv_p[sl])
        e = _edge_call(Ag, Bg, ep, s * S_EDGE)
        ps = _scatter_add_call(e, recv_p[sl])
        parts += [ps[0], ps[1]]

    A2, B2 = _node_call(h, parts, npar, Wc1[:D], Wc1[D:])

    outs = []
    for s in range(STRIPS):
        sl = slice(s * S_CLF, (s + 1) * S_CLF)
        A2g, B2g = _gather_pair_call(S_CLF, A2, B2, ei0_p[sl], ei1_p[sl])
        outs.append(_clf_call(A2g, B2g, row(bc1), Wc2, bc2.reshape(1, 1)))
    out = jnp.concatenate([o.reshape(-1) for o in outs])
    return out[:E]


# speedup vs baseline: 1.1434x; 1.0397x over previous
import functools

import jax
import jax.numpy as jnp
from jax import lax
from jax.experimental import pallas as pl
from jax.experimental.pallas import tpu as pltpu
from jax.experimental.pallas import tpu_sc as plsc

N_NODES = 10000
E = 320000
E2 = 2 * E
D = 128

NC = 2
NS = 16
NW = NC * NS
CB = 128


def _pad_to(n, m):
    return ((n + m - 1) // m) * m


STRIPS = 2
STRIPS_CLF = 4
P_EDGE = _pad_to(E2, STRIPS * NW * CB)
P_CLF = _pad_to(E, STRIPS_CLF * NW * CB)
S_EDGE = P_EDGE // STRIPS
S_CLF = P_CLF // STRIPS_CLF


def _dot(a, b):
    return jax.lax.dot_general(
        a.astype(jnp.bfloat16), b.astype(jnp.bfloat16),
        dimension_numbers=(((1,), (0,)), ((), ())),
        preferred_element_type=jnp.float32)


def _rowsum(t, o):
    th = t.astype(jnp.bfloat16)
    tl = (t - th.astype(jnp.float32)).astype(jnp.bfloat16)
    d = functools.partial(
        jax.lax.dot_general,
        dimension_numbers=(((1,), (0,)), ((), ())),
        preferred_element_type=jnp.float32)
    return d(th, o) + d(tl, o)


def _ln_relu(t, g, bt, o):
    mu = _rowsum(t, o)
    q = _rowsum(t * t, o)
    var = q - mu * mu
    rstd = lax.rsqrt(var + 1e-5)
    return jnp.maximum((t - mu) * rstd * g + bt, 0.0)


def _ln(t, g, bt):
    mu = jnp.mean(t, axis=-1, keepdims=True)
    var = jnp.mean((t - mu) ** 2, axis=-1, keepdims=True)
    return (t - mu) * lax.rsqrt(var + 1e-5) * g + bt


def _enc_body(x_ref, w1, b1, w2, b2, wa, wb, h_ref, a_ref, b_ref):
    h = jnp.maximum(_dot(x_ref[...], w1[...]) + b1[...], 0.0)
    h = jnp.maximum(_dot(h, w2[...]) + b2[...], 0.0)
    h_ref[...] = h
    a_ref[...] = _dot(h, wa[...])
    b_ref[...] = _dot(h, wb[...])


def _enc_call(x, W1, b1, W2, b2, Wa, Wb):
    RB = 1000
    wspec = pl.BlockSpec((D, D), lambda i: (0, 0))
    bspec = pl.BlockSpec((1, D), lambda i: (0, 0))
    rspec = pl.BlockSpec((RB, D), lambda i: (i, 0))
    return pl.pallas_call(
        _enc_body,
        grid=(N_NODES // RB,),
        in_specs=[rspec, wspec, bspec, wspec, bspec, wspec, wspec],
        out_specs=[rspec, rspec, rspec],
        out_shape=[jax.ShapeDtypeStruct((N_NODES, D), jnp.float32)] * 3,
    )(x, W1, b1, W2, b2, Wa, Wb)


def _make_edge_body(limit):
    def _edge_body(ag_ref, bg_ref, b1, g1, t1, w2, b2, g2, t2,
                   w3, b3, g3, t3, w4, b4, g4, t4, e_ref):
        RB = ag_ref.shape[0]
        t = ag_ref[...] + bg_ref[...] + b1[...]
        t = jnp.maximum(_ln(t, g1[...], t1[...]), 0.0)
        t = jnp.maximum(_ln(_dot(t, w2[...]) + b2[...], g2[...], t2[...]), 0.0)
        t = jnp.maximum(_ln(_dot(t, w3[...]) + b3[...], g3[...], t3[...]), 0.0)
        t = jnp.maximum(_ln(_dot(t, w4[...]) + b4[...], g4[...], t4[...]), 0.0)
        row = (pl.program_id(0) * RB
               + lax.broadcasted_iota(jnp.int32, (RB, 1), 0))
        e_ref[...] = jnp.where(row < limit, t, 0.0)
    return _edge_body


def _edge_call(ag, bg, ep, base):
    RB = 2048
    P = ag.shape[0]
    wspec = pl.BlockSpec((D, D), lambda i: (0, 0))
    bspec = pl.BlockSpec((1, D), lambda i: (0, 0))
    rspec = pl.BlockSpec((RB, D), lambda i: (i, 0))
    (w1_, b1, g1, t1), (w2, b2, g2, t2), (w3, b3, g3, t3), (w4, b4, g4, t4) = ep
    args = [ag, bg, b1, g1, t1, w2, b2, g2, t2,
            w3, b3, g3, t3, w4, b4, g4, t4]
    specs = [rspec, rspec, bspec, bspec, bspec, wspec, bspec, bspec,
             bspec, wspec, bspec, bspec, bspec, wspec, bspec, bspec, bspec]
    return pl.pallas_call(
        _make_edge_body(E2 - base),
        grid=(P // RB,),
        in_specs=specs,
        out_specs=rspec,
        out_shape=jax.ShapeDtypeStruct((P, D), jnp.float32),
    )(*args)


def _node_body(h_ref, p0_ref, p1_ref, p2_ref, p3_ref, wt, wb, b1, g1, t1,
               w2, b2, g2, t2, w3, b3, g3, t3, w4, b4, g4, t4, wct, wcb,
               a2_ref, b2_ref):
    h = h_ref[...]
    aggr = (p0_ref[...] + p1_ref[...]) + (p2_ref[...] + p3_ref[...])
    t = _dot(h, wt[...]) + _dot(aggr, wb[...]) + b1[...]
    t = jnp.maximum(_ln(t, g1[...], t1[...]), 0.0)
    t = jnp.maximum(_ln(_dot(t, w2[...]) + b2[...], g2[...], t2[...]), 0.0)
    t = jnp.maximum(_ln(_dot(t, w3[...]) + b3[...], g3[...], t3[...]), 0.0)
    t = jnp.maximum(_ln(_dot(t, w4[...]) + b4[...], g4[...], t4[...]), 0.0)
    h2 = t + h
    a2_ref[...] = _dot(h2, wct[...])
    b2_ref[...] = _dot(h2, wcb[...])


def _node_call(h, parts, np_, Wct, Wcb):
    RB = 1000
    wspec = pl.BlockSpec((D, D), lambda i: (0, 0))
    bspec = pl.BlockSpec((1, D), lambda i: (0, 0))
    rspec = pl.BlockSpec((RB, D), lambda i: (i, 0))
    (w1, b1, g1, t1), (w2, b2, g2, t2), (w3, b3, g3, t3), (w4, b4, g4, t4) = np_
    args = [h, *parts, w1[:D], w1[D:], b1, g1, t1, w2, b2, g2, t2,
            w3, b3, g3, t3, w4, b4, g4, t4, Wct, Wcb]
    specs = [rspec, rspec, rspec, rspec, rspec, wspec, wspec,
             bspec, bspec, bspec,
             wspec, bspec, bspec, bspec, wspec, bspec, bspec, bspec,
             wspec, bspec, bspec, bspec, wspec, wspec]
    return pl.pallas_call(
        _node_body,
        grid=(N_NODES // RB,),
        in_specs=specs,
        out_specs=[rspec, rspec],
        out_shape=[jax.ShapeDtypeStruct((N_NODES, D), jnp.float32)] * 2,
    )(*args)


def _clf_body(ag_ref, bg_ref, b1, w2r, b2, o_ref):
    t = jnp.maximum(ag_ref[...] + bg_ref[...] + b1[...], 0.0)
    tb = t.astype(jnp.bfloat16).astype(jnp.float32)
    wb = w2r[...].astype(jnp.bfloat16).astype(jnp.float32)
    y = jnp.sum(tb * wb, axis=1) + b2[0, 0]
    o_ref[...] = y[None, None, :]


def _clf_call(ag, bg, b1, W2, b2):
    RB = 2048
    P = ag.shape[0]
    rspec = pl.BlockSpec((RB, D), lambda i: (i, 0))
    return pl.pallas_call(
        _clf_body,
        grid=(P // RB,),
        in_specs=[rspec, rspec, pl.BlockSpec((1, D), lambda i: (0, 0)),
                  pl.BlockSpec((1, D), lambda i: (0, 0)),
                  pl.BlockSpec((1, 1), lambda i: (0, 0))],
        out_specs=pl.BlockSpec((1, 1, RB), lambda i: (i, 0, 0)),
        out_shape=jax.ShapeDtypeStruct((P // RB, 1, RB), jnp.float32),
    )(ag, bg, b1, W2.reshape(1, -1), b2)


def _gather_pair_call(P, a, b, ia, ib):
    per_w = P // NW
    n_chunks = per_w // CB
    dt = a.dtype
    W = a.shape[1]
    mesh = plsc.VectorSubcoreMesh(core_axis_name="c", subcore_axis_name="s")

    @functools.partial(
        pl.kernel,
        out_type=(jax.ShapeDtypeStruct((P, W), dt),
                  jax.ShapeDtypeStruct((P, W), dt)),
        mesh=mesh,
        scratch_types=[pltpu.VMEM((CB,), jnp.int32),
                       pltpu.VMEM((CB,), jnp.int32),
                       pltpu.VMEM((CB, W), dt),
                       pltpu.VMEM((CB, W), dt),
                       pltpu.SemaphoreType.DMA,
                       pltpu.SemaphoreType.DMA],
    )
    def gather2(a_hbm, b_hbm, ia_hbm, ib_hbm, oa_hbm, ob_hbm,
                ia_v, ib_v, ra_v, rb_v, sa, sb):
        wid = lax.axis_index("s") * NC + lax.axis_index("c")
        base0 = wid * per_w

        @pl.loop(0, n_chunks)
        def _(i):
            base = base0 + i * CB
            pltpu.sync_copy(ia_hbm.at[pl.ds(base, CB)], ia_v)
            pltpu.sync_copy(ib_hbm.at[pl.ds(base, CB)], ib_v)
            ca = pltpu.async_copy(a_hbm.at[ia_v], ra_v, sa)
            cb = pltpu.async_copy(b_hbm.at[ib_v], rb_v, sb)
            ca.wait()
            cb.wait()
            pltpu.sync_copy(ra_v, oa_hbm.at[pl.ds(base, CB)])
            pltpu.sync_copy(rb_v, ob_hbm.at[pl.ds(base, CB)])

    return gather2(a, b, ia, ib)


def _scatter_add_call(e, idx):
    per_core = e.shape[0] // NC
    per_sub = per_core // NS
    n_chunks = per_sub // CB
    RPS = 624
    LAST = N_NODES - (NS - 1) * RPS
    ZR = 16
    mesh = plsc.VectorSubcoreMesh(core_axis_name="c", subcore_axis_name="s")

    @functools.partial(
        pl.kernel,
        out_type=jax.ShapeDtypeStruct((NC, N_NODES, D), jnp.float32),
        mesh=mesh,
        scratch_types=[pltpu.VMEM((CB,), jnp.int32),
                       pltpu.VMEM((CB, D), jnp.float32),
                       pltpu.VMEM((ZR, D), jnp.float32),
                       pltpu.VMEM_SHARED((N_NODES, D), jnp.float32)],
    )
    def scatter_add(e_hbm, idx_hbm, out_hbm, idx_v, rows_v, z_v, acc_sh):
        cid = lax.axis_index("c")
        sid = lax.axis_index("s")

        @pl.loop(0, ZR)
        def _(r):
            @pl.loop(0, D // 16)
            def _(k):
                z_v[r, pl.ds(k * 16, 16)] = jnp.zeros((16,), jnp.float32)

        @pl.loop(0, LAST // ZR)
        def _(j):
            pltpu.sync_copy(z_v, acc_sh.at[pl.ds(sid * RPS + j * ZR, ZR)])

        plsc.subcore_barrier()
        base0 = cid * per_core + sid * per_sub

        @pl.loop(0, n_chunks)
        def _(i):
            base = base0 + i * CB
            pltpu.sync_copy(idx_hbm.at[pl.ds(base, CB)], idx_v)
            pltpu.sync_copy(e_hbm.at[pl.ds(base, CB)], rows_v)
            pltpu.sync_copy(rows_v, acc_sh.at[idx_v], add=True)

        plsc.subcore_barrier()

        @pl.when(sid != NS - 1)
        def _():
            pltpu.sync_copy(acc_sh.at[pl.ds(sid * RPS, RPS)],
                            out_hbm.at[cid].at[pl.ds(sid * RPS, RPS)])

        @pl.when(sid == NS - 1)
        def _():
            pltpu.sync_copy(acc_sh.at[pl.ds((NS - 1) * RPS, LAST)],
                            out_hbm.at[cid].at[pl.ds((NS - 1) * RPS, LAST)])

    return scatter_add(e, idx)


def kernel(x, edge_index, enc_params, edge_params, node_params, clf_params):
    ei0, ei1 = edge_index[0], edge_index[1]
    send = jnp.concatenate([ei0, ei1])
    recv = jnp.concatenate([ei1, ei0])
    send_p = jnp.pad(send, (0, P_EDGE - E2))
    recv_p = jnp.pad(recv, (0, P_EDGE - E2))
    ei0_p = jnp.pad(ei0, (0, P_CLF - E))
    ei1_p = jnp.pad(ei1, (0, P_CLF - E))

    (W1e, b1e, _, _), (W2e, b2e, _, _) = enc_params
    We1 = edge_params[0][0]
    row = lambda v: v.reshape(1, -1)

    ep = [(w, row(b), row(g), row(bt)) for (w, b, g, bt) in edge_params]
    npar = [(w, row(b), row(g), row(bt)) for (w, b, g, bt) in node_params]
    (Wc1, bc1, _, _), (Wc2, bc2, _, _) = clf_params

    h, A, B = _enc_call(x, W1e, row(b1e), W2e, row(b2e), We1[:D], We1[D:])

    parts = []
    for s in range(STRIPS):
        sl = slice(s * S_EDGE, (s + 1) * S_EDGE)
        Ag, Bg = _gather_pair_call(S_EDGE, A, B, send_p[sl], recv_p[sl])
        e = _edge_call(Ag, Bg, ep, s * S_EDGE)
        ps = _scatter_add_call(e, recv_p[sl])
        parts += [ps[0], ps[1]]

    A2, B2 = _node_call(h, parts, npar, Wc1[:D], Wc1[D:])

    outs = []
    for s in range(STRIPS_CLF):
        sl = slice(s * S_CLF, (s + 1) * S_CLF)
        A2g, B2g = _gather_pair_call(S_CLF, A2, B2, ei0_p[sl], ei1_p[sl])
        outs.append(_clf_call(A2g, B2g, row(bc1), Wc2, bc2.reshape(1, 1)))
    out = jnp.concatenate([o.reshape(-1) for o in outs])
    return out[:E]
